# baseline (device time: 90406 ns/iter reference)
import jax
import jax.numpy as jnp
from jax import lax
from jax.experimental import pallas as pl
from jax.experimental.pallas import tpu as pltpu

N_DEV = 16


def kernel(x, w_mat):
    m_tot, k_blk = x.shape
    k_tot, n_tot = w_mat.shape
    m_blk = m_tot // N_DEV
    n_steps = N_DEV // 2
    assert k_tot == N_DEV * k_blk

    my_idx = lax.axis_index("i")
    sched = jnp.mod(
        my_idx - jnp.arange(N_DEV, dtype=jnp.int32), N_DEV
    ).astype(jnp.int32)

    def body(sched_ref, x_ref, w1_ref, w2_ref, out_ref,
             recv_buf, send_sems, recv_sems):
        t = pl.program_id(0)
        my = lax.axis_index("i")

        @pl.when(t == 0)
        def _prologue():
            pltpu.make_async_copy(
                x_ref.at[pl.ds(my * m_blk, m_blk), :],
                recv_buf.at[my],
                recv_sems.at[my],
            ).start()
            for o in range(1, N_DEV):
                d = lax.rem(my + o, N_DEV)
                pltpu.make_async_remote_copy(
                    src_ref=x_ref.at[pl.ds(d * m_blk, m_blk), :],
                    dst_ref=recv_buf.at[my],
                    send_sem=send_sems.at[o],
                    recv_sem=recv_sems.at[my],
                    device_id=(d,),
                    device_id_type=pl.DeviceIdType.MESH,
                ).start()

        j1 = sched_ref[2 * t]
        j2 = sched_ref[2 * t + 1]

        for j in (j1, j2):
            pltpu.make_async_remote_copy(
                src_ref=x_ref.at[pl.ds(0, m_blk), :],
                dst_ref=recv_buf.at[j],
                send_sem=send_sems.at[0],
                recv_sem=recv_sems.at[j],
                device_id=(my,),
                device_id_type=pl.DeviceIdType.MESH,
            ).wait_recv()

        c = jnp.dot(
            recv_buf[j1], w1_ref[...], preferred_element_type=jnp.float32
        ) + jnp.dot(
            recv_buf[j2], w2_ref[...], preferred_element_type=jnp.float32
        )

        @pl.when(t == 0)
        def _():
            out_ref[...] = c

        @pl.when(jnp.logical_and(t > 0, t < n_steps - 1))
        def _():
            out_ref[...] += c

        @pl.when(t == n_steps - 1)
        def _():
            y = out_ref[...] + c
            out_ref[...] = y * jax.nn.sigmoid(y)
            for o in range(1, N_DEV):
                pltpu.make_async_remote_copy(
                    src_ref=x_ref.at[pl.ds(0, m_blk), :],
                    dst_ref=recv_buf.at[0],
                    send_sem=send_sems.at[o],
                    recv_sem=recv_sems.at[0],
                    device_id=(my,),
                    device_id_type=pl.DeviceIdType.MESH,
                ).wait_send()

    grid_spec = pltpu.PrefetchScalarGridSpec(
        num_scalar_prefetch=1,
        grid=(n_steps,),
        in_specs=[
            pl.BlockSpec((m_tot, k_blk), lambda t, s: (0, 0)),
            pl.BlockSpec((k_blk, n_tot), lambda t, s: (s[2 * t], 0)),
            pl.BlockSpec((k_blk, n_tot), lambda t, s: (s[2 * t + 1], 0)),
        ],
        out_specs=pl.BlockSpec((m_blk, n_tot), lambda t, s: (0, 0)),
        scratch_shapes=[
            pltpu.VMEM((N_DEV, m_blk, k_blk), jnp.float32),
            pltpu.SemaphoreType.DMA((N_DEV,)),
            pltpu.SemaphoreType.DMA((N_DEV,)),
        ],
    )
    return pl.pallas_call(
        body,
        grid_spec=grid_spec,
        out_shape=jax.ShapeDtypeStruct((m_blk, n_tot), jnp.float32),
        compiler_params=pltpu.CompilerParams(
            dimension_semantics=("arbitrary",),
            vmem_limit_bytes=60 * 1024 * 1024,
        ),
    )(sched, x, w_mat, w_mat)


# device time: 76302 ns/iter; 1.1848x vs baseline; 1.1848x over previous
import jax
import jax.numpy as jnp
from jax import lax
from jax.experimental import pallas as pl
from jax.experimental.pallas import tpu as pltpu

N_DEV = 16


def kernel(x, w_mat):
    m_tot, k_blk = x.shape
    k_tot, n_tot = w_mat.shape
    m_blk = m_tot // N_DEV
    assert k_tot == N_DEV * k_blk

    def wmap(t):
        return (lax.rem(lax.axis_index("i") - t + N_DEV, N_DEV), 0)

    def body(x_ref, w_ref, out_ref, x_bf, recv_buf, send_sems, recv_sems):
        t = pl.program_id(0)
        my = lax.axis_index("i")
        j = lax.rem(my - t + N_DEV, N_DEV)

        @pl.when(t == 0)
        def _prologue():
            x_bf[...] = x_ref[...].astype(jnp.bfloat16)
            pltpu.make_async_copy(
                x_bf.at[pl.ds(my * m_blk, m_blk), :],
                recv_buf.at[my],
                recv_sems.at[my],
            ).start()
            for o in range(1, N_DEV):
                d = lax.rem(my + o, N_DEV)
                pltpu.make_async_remote_copy(
                    src_ref=x_bf.at[pl.ds(d * m_blk, m_blk), :],
                    dst_ref=recv_buf.at[my],
                    send_sem=send_sems.at[o],
                    recv_sem=recv_sems.at[my],
                    device_id=(d,),
                    device_id_type=pl.DeviceIdType.MESH,
                ).start()

        pltpu.make_async_remote_copy(
            src_ref=x_bf.at[pl.ds(0, m_blk), :],
            dst_ref=recv_buf.at[j],
            send_sem=send_sems.at[0],
            recv_sem=recv_sems.at[j],
            device_id=(my,),
            device_id_type=pl.DeviceIdType.MESH,
        ).wait_recv()

        c = jnp.dot(
            recv_buf[j].astype(jnp.float32), w_ref[...],
            preferred_element_type=jnp.float32,
        )

        @pl.when(t == 0)
        def _():
            out_ref[...] = c

        @pl.when(jnp.logical_and(t > 0, t < N_DEV - 1))
        def _():
            out_ref[...] += c

        @pl.when(t == N_DEV - 1)
        def _():
            y = out_ref[...] + c
            out_ref[...] = y * jax.nn.sigmoid(y)
            for o in range(1, N_DEV):
                pltpu.make_async_remote_copy(
                    src_ref=x_bf.at[pl.ds(0, m_blk), :],
                    dst_ref=recv_buf.at[0],
                    send_sem=send_sems.at[o],
                    recv_sem=recv_sems.at[0],
                    device_id=(my,),
                    device_id_type=pl.DeviceIdType.MESH,
                ).wait_send()

    return pl.pallas_call(
        body,
        grid=(N_DEV,),
        in_specs=[
            pl.BlockSpec((m_tot, k_blk), lambda t: (0, 0)),
            pl.BlockSpec((k_blk, n_tot), wmap),
        ],
        out_specs=pl.BlockSpec((m_blk, n_tot), lambda t: (0, 0)),
        out_shape=jax.ShapeDtypeStruct((m_blk, n_tot), jnp.float32),
        scratch_shapes=[
            pltpu.VMEM((m_tot, k_blk), jnp.bfloat16),
            pltpu.VMEM((N_DEV, m_blk, k_blk), jnp.bfloat16),
            pltpu.SemaphoreType.DMA((N_DEV,)),
            pltpu.SemaphoreType.DMA((N_DEV,)),
        ],
        compiler_params=pltpu.CompilerParams(
            dimension_semantics=("arbitrary",),
            vmem_limit_bytes=60 * 1024 * 1024,
        ),
    )(x, w_mat)


# device time: 74230 ns/iter; 1.2179x vs baseline; 1.0279x over previous
import jax
import jax.numpy as jnp
from jax import lax
from jax.experimental import pallas as pl
from jax.experimental.pallas import tpu as pltpu

N_DEV = 16


def kernel(x, w_mat):
    m_tot, k_blk = x.shape
    k_tot, n_tot = w_mat.shape
    m_blk = m_tot // N_DEV
    assert k_tot == N_DEV * k_blk

    def wmap(t):
        return (lax.rem(lax.axis_index("i") - t + N_DEV, N_DEV), 0)

    def body(x_ref, w_ref, out_ref, x_bf, recv_buf, send_sems, recv_sems):
        t = pl.program_id(0)
        my = lax.axis_index("i")
        j = lax.rem(my - t + N_DEV, N_DEV)

        @pl.when(t == 0)
        def _prologue():
            x_bf[...] = x_ref[...].astype(jnp.bfloat16)
            pltpu.make_async_copy(
                x_bf.at[pl.ds(my * m_blk, m_blk), :],
                recv_buf.at[my],
                recv_sems.at[my],
            ).start()

        LOOKAHEAD = 3
        for o in range(1, N_DEV):
            @pl.when(t == max(0, o - LOOKAHEAD))
            def _send(o=o):
                d = lax.rem(my + o, N_DEV)
                pltpu.make_async_remote_copy(
                    src_ref=x_bf.at[pl.ds(d * m_blk, m_blk), :],
                    dst_ref=recv_buf.at[my],
                    send_sem=send_sems.at[o],
                    recv_sem=recv_sems.at[my],
                    device_id=(d,),
                    device_id_type=pl.DeviceIdType.MESH,
                ).start()

        pltpu.make_async_remote_copy(
            src_ref=x_bf.at[pl.ds(0, m_blk), :],
            dst_ref=recv_buf.at[j],
            send_sem=send_sems.at[0],
            recv_sem=recv_sems.at[j],
            device_id=(my,),
            device_id_type=pl.DeviceIdType.MESH,
        ).wait_recv()

        c = jnp.dot(
            recv_buf[j].astype(jnp.float32), w_ref[...],
            preferred_element_type=jnp.float32,
        )

        @pl.when(t == 0)
        def _():
            out_ref[...] = c

        @pl.when(jnp.logical_and(t > 0, t < N_DEV - 1))
        def _():
            out_ref[...] += c

        @pl.when(t == N_DEV - 1)
        def _():
            y = out_ref[...] + c
            out_ref[...] = y * jax.nn.sigmoid(y)
            for o in range(1, N_DEV):
                pltpu.make_async_remote_copy(
                    src_ref=x_bf.at[pl.ds(0, m_blk), :],
                    dst_ref=recv_buf.at[0],
                    send_sem=send_sems.at[o],
                    recv_sem=recv_sems.at[0],
                    device_id=(my,),
                    device_id_type=pl.DeviceIdType.MESH,
                ).wait_send()

    return pl.pallas_call(
        body,
        grid=(N_DEV,),
        in_specs=[
            pl.BlockSpec((m_tot, k_blk), lambda t: (0, 0)),
            pl.BlockSpec((k_blk, n_tot), wmap),
        ],
        out_specs=pl.BlockSpec((m_blk, n_tot), lambda t: (0, 0)),
        out_shape=jax.ShapeDtypeStruct((m_blk, n_tot), jnp.float32),
        scratch_shapes=[
            pltpu.VMEM((m_tot, k_blk), jnp.bfloat16),
            pltpu.VMEM((N_DEV, m_blk, k_blk), jnp.bfloat16),
            pltpu.SemaphoreType.DMA((N_DEV,)),
            pltpu.SemaphoreType.DMA((N_DEV,)),
        ],
        compiler_params=pltpu.CompilerParams(
            dimension_semantics=("arbitrary",),
            vmem_limit_bytes=60 * 1024 * 1024,
        ),
    )(x, w_mat)
